# baseline (speedup 1.0000x reference)
import jax
import jax.numpy as jnp
from jax.experimental import pallas as pl
from jax.experimental.pallas import tpu as pltpu

E = 8
DIM = 1024
INTER = 2048
LT_DIM = 768
TOKENS = 2048
SH = 2 * INTER

TM = 256
TN = 512
NT = TOKENS // TM
NK = INTER // TN
NKS = SH // TN


def _silu(v):
    return v * jax.nn.sigmoid(v)


def _routed_body(x_ref, lt_ref, emb_ref, ew_ref,
                 w1_ref, b1_ref, w3_ref, b3_ref, w2_ref, b2_ref,
                 out_ref, mask_ref):
    e = pl.program_id(1)
    k = pl.program_id(2)

    @pl.when(jnp.logical_and(e == 0, k == 0))
    def _init():
        out_ref[...] = jnp.zeros_like(out_ref)

    @pl.when(k == 0)
    def _route():
        emb = emb_ref[...]
        emb = emb / jnp.sqrt(jnp.sum(emb * emb, axis=-1, keepdims=True) + 1e-24)
        sims = jax.lax.dot_general(lt_ref[...], emb, (((1,), (1,)), ((), ())),
                                   preferred_element_type=jnp.float32)
        idx = jnp.argmax(sims, axis=-1)
        mask_ref[...] = (idx[:, None] == e).astype(jnp.float32)

    x = x_ref[...]
    a = jax.lax.dot_general(x, w1_ref[0], (((1,), (0,)), ((), ())),
                            preferred_element_type=jnp.float32) + b1_ref[0, 0]
    g = jax.lax.dot_general(x, w3_ref[0], (((1,), (0,)), ((), ())),
                            preferred_element_type=jnp.float32) + b3_ref[0, 0]
    h = _silu(a) * g
    o = jax.lax.dot_general(h, w2_ref[0], (((1,), (0,)), ((), ())),
                            preferred_element_type=jnp.float32)
    scale = mask_ref[...] * ew_ref[e]

    @pl.when(k == 0)
    def _with_bias():
        out_ref[...] += scale * (o + b2_ref[0, 0])

    @pl.when(k != 0)
    def _no_bias():
        out_ref[...] += scale * o


def _shared_body(x_ref, y_ref, ws1_ref, bs1_ref, ws2_ref, bs2_ref, out_ref):
    k = pl.program_id(1)

    @pl.when(k == 0)
    def _init():
        out_ref[...] = y_ref[...] + bs2_ref[...]

    x = x_ref[...]
    a = jax.lax.dot_general(x, ws1_ref[...], (((1,), (0,)), ((), ())),
                            preferred_element_type=jnp.float32) + bs1_ref[...]
    out_ref[...] += jax.lax.dot_general(_silu(a), ws2_ref[...],
                                        (((1,), (0,)), ((), ())),
                                        preferred_element_type=jnp.float32)


def kernel(x, language_token, routing_embeddings, expert_weights,
           w1, b1, w2, b2, w3, b3, ws1, bs1, ws2, bs2):
    y = pl.pallas_call(
        _routed_body,
        grid=(NT, E, NK),
        in_specs=[
            pl.BlockSpec((TM, DIM), lambda t, e, k: (t, 0)),
            pl.BlockSpec((TM, LT_DIM), lambda t, e, k: (t, 0)),
            pl.BlockSpec((E, LT_DIM), lambda t, e, k: (0, 0)),
            pl.BlockSpec(memory_space=pltpu.SMEM),
            pl.BlockSpec((1, DIM, TN), lambda t, e, k: (e, 0, k)),
            pl.BlockSpec((1, 1, TN), lambda t, e, k: (e, 0, k)),
            pl.BlockSpec((1, DIM, TN), lambda t, e, k: (e, 0, k)),
            pl.BlockSpec((1, 1, TN), lambda t, e, k: (e, 0, k)),
            pl.BlockSpec((1, TN, DIM), lambda t, e, k: (e, k, 0)),
            pl.BlockSpec((1, 1, DIM), lambda t, e, k: (e, 0, 0)),
        ],
        out_specs=pl.BlockSpec((TM, DIM), lambda t, e, k: (t, 0)),
        out_shape=jax.ShapeDtypeStruct((TOKENS, DIM), jnp.float32),
        scratch_shapes=[pltpu.VMEM((TM, 1), jnp.float32)],
    )(x, language_token, routing_embeddings, expert_weights,
      w1, b1.reshape(E, 1, INTER), w3, b3.reshape(E, 1, INTER),
      w2, b2.reshape(E, 1, DIM))

    out = pl.pallas_call(
        _shared_body,
        grid=(NT, NKS),
        in_specs=[
            pl.BlockSpec((TM, DIM), lambda t, k: (t, 0)),
            pl.BlockSpec((TM, DIM), lambda t, k: (t, 0)),
            pl.BlockSpec((DIM, TN), lambda t, k: (0, k)),
            pl.BlockSpec((1, TN), lambda t, k: (0, k)),
            pl.BlockSpec((TN, DIM), lambda t, k: (k, 0)),
            pl.BlockSpec((1, DIM), lambda t, k: (0, 0)),
        ],
        out_specs=pl.BlockSpec((TM, DIM), lambda t, k: (t, 0)),
        out_shape=jax.ShapeDtypeStruct((TOKENS, DIM), jnp.float32),
    )(x, y, ws1, bs1.reshape(1, SH), ws2, bs2.reshape(1, DIM))
    return out


# speedup vs baseline: 1.4339x; 1.4339x over previous
import functools

import jax
import jax.numpy as jnp
from jax import lax
from jax.experimental import pallas as pl
from jax.experimental.pallas import tpu as pltpu
from jax.experimental.pallas import tpu_sc as plsc

E = 8
DIM = 1024
INTER = 2048
LT_DIM = 768
TOKENS = 2048
SH = 2 * INTER

TM = 256
TN = 512
P = TOKENS + E * TM
NTP = P // TM
NK = INTER // TN
NTS = TOKENS // TM
NKS = SH // TN
TRASH = TOKENS

NC = 2
NS = 16
NW = NC * NS
TPW = TOKENS // NS
SPW = P // NW


def _silu(v):
    return v * jax.nn.sigmoid(v)


def _route_body(lt_ref, emb_ref, idx_ref):
    emb = emb_ref[...]
    enorm = jnp.sqrt(jnp.sum(emb * emb, axis=-1, keepdims=True))
    emb = emb / jnp.maximum(enorm, 1e-12)
    lt = lt_ref[...]
    tnorm = jnp.sqrt(jnp.sum(lt * lt, axis=-1, keepdims=True))
    lt = lt / jnp.maximum(tnorm, 1e-12)
    sims = lax.dot_general(lt, emb, (((1,), (1,)), ((), ())),
                           preferred_element_type=jnp.float32)
    idx_ref[...] = jnp.argmax(sims, axis=-1).astype(jnp.int32)


def _route(language_token, routing_embeddings):
    return pl.pallas_call(
        _route_body,
        grid=(1,),
        in_specs=[
            pl.BlockSpec((TOKENS, LT_DIM), lambda i: (0, 0)),
            pl.BlockSpec((E, LT_DIM), lambda i: (0, 0)),
        ],
        out_specs=pl.BlockSpec((TOKENS,), lambda i: (0,)),
        out_shape=jax.ShapeDtypeStruct((TOKENS,), jnp.int32),
    )(language_token, routing_embeddings)


PPW = P // NW
NCHUNK = TOKENS // 16


def _sort_body(idx_hbm, perm_hbm, texp_hbm, tvalid_hbm,
               idx_v, chunk_v, meta_v, sem):
    cid = lax.axis_index("c")
    sid = lax.axis_index("s")
    w = sid * NC + cid
    base = w * PPW
    pltpu.sync_copy(idx_hbm, idx_v)
    lanes = lax.iota(jnp.int32, 16)

    def hist_step(i, counts):
        vec = idx_v[pl.ds(i * 16, 16)]
        for e in range(E):
            m = (vec == e).astype(jnp.int32)
            counts = counts + jnp.where(lanes == e, m * 0 + jnp.full(
                (16,), jnp.sum(m), jnp.int32), jnp.zeros((16,), jnp.int32))
        return counts
    totals = lax.fori_loop(0, NCHUNK, hist_step, jnp.zeros((16,), jnp.int32))

    padded = jnp.bitwise_and(totals + (TM - 1), -TM)
    incl = plsc.cumsum(padded)
    seg = incl - padded
    total_padded = jnp.max(incl)

    for i in range(PPW // 16):
        chunk_v[pl.ds(i * 16, 16)] = jnp.full((16,), TRASH, jnp.int32)
    lo = jnp.full((16,), base, jnp.int32)
    hi = jnp.full((16,), base + PPW, jnp.int32)

    def slot_step(i, cur):
        vec = idx_v[pl.ds(i * 16, 16)]
        svec = jnp.zeros((16,), jnp.int32)
        ncur = []
        for e in range(E):
            m = vec == e
            pos = plsc.cumsum(m.astype(jnp.int32))
            cvec = jnp.full((16,), cur[e], jnp.int32)
            svec = jnp.where(m, cvec + pos - 1, svec)
            ncur.append(cur[e] + pos[15])
        ids = lanes + jnp.full((16,), i * 16, jnp.int32)
        keep = jnp.logical_and(svec >= lo, svec < hi)
        plsc.store_scatter(chunk_v, [svec - lo], ids, mask=keep)
        return tuple(ncur)
    lax.fori_loop(0, NCHUNK, slot_step,
                  tuple(seg[e] for e in range(E)), unroll=2)

    pltpu.sync_copy(chunk_v, perm_hbm.at[pl.ds(base, PPW)])

    @pl.when(w == 0)
    def _meta():
        tp = jnp.full((16,), total_padded, jnp.int32)
        tile_starts = lax.iota(jnp.int32, 16) * TM
        valid = (tile_starts < tp).astype(jnp.int32)
        eff = jnp.minimum(tile_starts, tp - TM)
        acc = jnp.full((16,), -1, jnp.int32)
        for e in range(E):
            acc = acc + (eff >= jnp.full((16,), seg[e], jnp.int32)
                         ).astype(jnp.int32)
        meta_v[...] = acc
        pltpu.sync_copy(meta_v, texp_hbm)
        meta_v[...] = valid
        pltpu.sync_copy(meta_v, tvalid_hbm)


def _sort(idx):
    mesh = plsc.VectorSubcoreMesh(core_axis_name="c", subcore_axis_name="s",
                                  num_cores=NC, num_subcores=NS)
    return pl.kernel(
        _sort_body,
        out_type=(
            jax.ShapeDtypeStruct((P,), jnp.int32),
            jax.ShapeDtypeStruct((16,), jnp.int32),
            jax.ShapeDtypeStruct((16,), jnp.int32),
        ),
        mesh=mesh,
        compiler_params=pltpu.CompilerParams(needs_layout_passes=False),
        scratch_types=[
            pltpu.VMEM((TOKENS,), jnp.int32),
            pltpu.VMEM((PPW,), jnp.int32),
            pltpu.VMEM((16,), jnp.int32),
            pltpu.SemaphoreType.DMA,
        ],
    )(idx)


def _gather_body(perm_hbm, x_hbm, xs_hbm, ia_v, ib_v, rows_v, sem):
    cid = lax.axis_index("c")
    sid = lax.axis_index("s")
    w = sid * NC + cid
    base = w * SPW
    pltpu.sync_copy(perm_hbm.at[pl.ds(base, SPW // 2)], ia_v)
    pltpu.sync_copy(perm_hbm.at[pl.ds(base + SPW // 2, SPW // 2)], ib_v)
    for half, iv in ((0, ia_v), (1, ib_v)):
        for i in range(SPW // 2 // 16):
            iv[pl.ds(i * 16, 16)] = jnp.minimum(iv[pl.ds(i * 16, 16)],
                                                TOKENS - 1)
        pltpu.async_copy(x_hbm.at[iv], rows_v, sem).wait()
        pltpu.sync_copy(rows_v,
                        xs_hbm.at[pl.ds(base + half * (SPW // 2), SPW // 2)])


def _gather(perm, x):
    mesh = plsc.VectorSubcoreMesh(core_axis_name="c", subcore_axis_name="s",
                                  num_cores=NC, num_subcores=NS)
    return pl.kernel(
        _gather_body,
        out_type=jax.ShapeDtypeStruct((P, DIM), jnp.float32),
        mesh=mesh,
        compiler_params=pltpu.CompilerParams(needs_layout_passes=False),
        scratch_types=[
            pltpu.VMEM((SPW // 2,), jnp.int32),
            pltpu.VMEM((SPW // 2,), jnp.int32),
            pltpu.VMEM((SPW // 2, DIM), jnp.float32),
            pltpu.SemaphoreType.DMA,
        ],
    )(perm, x)


def _gmm_body(texp_ref, tvalid_ref, xs_ref, ew_ref,
              w1_ref, b1_ref, w3_ref, b3_ref, w2_ref, b2_ref, out_ref):
    t = pl.program_id(0)
    k = pl.program_id(1)

    @pl.when(k == 0)
    def _init():
        out_ref[...] = jnp.zeros_like(out_ref)

    @pl.when(tvalid_ref[t] == 1)
    def _compute():
        scale = ew_ref[texp_ref[t]]
        x = xs_ref[...]
        a = lax.dot_general(x, w1_ref[0], (((1,), (0,)), ((), ())),
                            preferred_element_type=jnp.float32) + b1_ref[0, 0]
        g = lax.dot_general(x, w3_ref[0], (((1,), (0,)), ((), ())),
                            preferred_element_type=jnp.float32) + b3_ref[0, 0]
        h = _silu(a) * g
        o = lax.dot_general(h, w2_ref[0], (((1,), (0,)), ((), ())),
                            preferred_element_type=jnp.float32)

        @pl.when(k == 0)
        def _wb():
            out_ref[...] += scale * (o + b2_ref[0, 0])

        @pl.when(k != 0)
        def _nb():
            out_ref[...] += scale * o


def _gmm(texp, tvalid, xs, expert_weights, w1, b1, w3, b3, w2, b2):
    grid_spec = pltpu.PrefetchScalarGridSpec(
        num_scalar_prefetch=2,
        grid=(NTP, NK),
        in_specs=[
            pl.BlockSpec((TM, DIM), lambda t, k, te, tv: (t, 0)),
            pl.BlockSpec(memory_space=pltpu.SMEM),
            pl.BlockSpec((1, DIM, TN), lambda t, k, te, tv: (te[t], 0, k)),
            pl.BlockSpec((1, 1, TN), lambda t, k, te, tv: (te[t], 0, k)),
            pl.BlockSpec((1, DIM, TN), lambda t, k, te, tv: (te[t], 0, k)),
            pl.BlockSpec((1, 1, TN), lambda t, k, te, tv: (te[t], 0, k)),
            pl.BlockSpec((1, TN, DIM), lambda t, k, te, tv: (te[t], k, 0)),
            pl.BlockSpec((1, 1, DIM), lambda t, k, te, tv: (te[t], 0, 0)),
        ],
        out_specs=pl.BlockSpec((TM, DIM), lambda t, k, te, tv: (t, 0)),
    )
    return pl.pallas_call(
        _gmm_body,
        grid_spec=grid_spec,
        out_shape=jax.ShapeDtypeStruct((P, DIM), jnp.float32),
    )(texp, tvalid, xs, expert_weights,
      w1, b1.reshape(E, 1, INTER), w3, b3.reshape(E, 1, INTER),
      w2, b2.reshape(E, 1, DIM))


def _scatter_body(ys_hbm, perm_hbm, yb_hbm, ia_v, ib_v, rows_v, sem):
    cid = lax.axis_index("c")
    sid = lax.axis_index("s")
    w = sid * NC + cid
    base = w * SPW
    pltpu.sync_copy(perm_hbm.at[pl.ds(base, SPW // 2)], ia_v)
    pltpu.sync_copy(perm_hbm.at[pl.ds(base + SPW // 2, SPW // 2)], ib_v)
    for half, iv in ((0, ia_v), (1, ib_v)):
        pltpu.sync_copy(ys_hbm.at[pl.ds(base + half * (SPW // 2), SPW // 2)],
                        rows_v)
        pltpu.async_copy(rows_v, yb_hbm.at[iv], sem).wait()


def _scatter(y_sorted, perm):
    mesh = plsc.VectorSubcoreMesh(core_axis_name="c", subcore_axis_name="s",
                                  num_cores=NC, num_subcores=NS)
    return pl.kernel(
        _scatter_body,
        out_type=jax.ShapeDtypeStruct((TOKENS + 8, DIM), jnp.float32),
        mesh=mesh,
        compiler_params=pltpu.CompilerParams(needs_layout_passes=False),
        scratch_types=[
            pltpu.VMEM((SPW // 2,), jnp.int32),
            pltpu.VMEM((SPW // 2,), jnp.int32),
            pltpu.VMEM((SPW // 2, DIM), jnp.float32),
            pltpu.SemaphoreType.DMA,
        ],
    )(y_sorted, perm)


def _shared_body(x_ref, y_ref, ws1_ref, bs1_ref, ws2_ref, bs2_ref, out_ref):
    k = pl.program_id(1)

    @pl.when(k == 0)
    def _init():
        out_ref[...] = y_ref[...] + bs2_ref[...]

    x = x_ref[...]
    a = lax.dot_general(x, ws1_ref[...], (((1,), (0,)), ((), ())),
                        preferred_element_type=jnp.float32) + bs1_ref[...]
    out_ref[...] += lax.dot_general(_silu(a), ws2_ref[...],
                                    (((1,), (0,)), ((), ())),
                                    preferred_element_type=jnp.float32)


def _shared(x, ybuf, ws1, bs1, ws2, bs2):
    return pl.pallas_call(
        _shared_body,
        grid=(NTS, NKS),
        in_specs=[
            pl.BlockSpec((TM, DIM), lambda t, k: (t, 0)),
            pl.BlockSpec((TM, DIM), lambda t, k: (t, 0)),
            pl.BlockSpec((DIM, TN), lambda t, k: (0, k)),
            pl.BlockSpec((1, TN), lambda t, k: (0, k)),
            pl.BlockSpec((TN, DIM), lambda t, k: (k, 0)),
            pl.BlockSpec((1, DIM), lambda t, k: (0, 0)),
        ],
        out_specs=pl.BlockSpec((TM, DIM), lambda t, k: (t, 0)),
        out_shape=jax.ShapeDtypeStruct((TOKENS, DIM), jnp.float32),
    )(x, ybuf, ws1, bs1.reshape(1, SH), ws2, bs2.reshape(1, DIM))


def kernel(x, language_token, routing_embeddings, expert_weights,
           w1, b1, w2, b2, w3, b3, ws1, bs1, ws2, bs2):
    idx = _route(language_token, routing_embeddings)
    perm, texp, tvalid = _sort(idx)
    xs = _gather(perm, x)
    ys = _gmm(texp, tvalid, xs, expert_weights, w1, b1, w3, b3, w2, b2)
    ybuf = _scatter(ys, perm)
    return _shared(x, ybuf, ws1, bs1, ws2, bs2)


# speedup vs baseline: 1.9888x; 1.3870x over previous
import functools

import jax
import jax.numpy as jnp
from jax import lax
from jax.experimental import pallas as pl
from jax.experimental.pallas import tpu as pltpu
from jax.experimental.pallas import tpu_sc as plsc

E = 8
DIM = 1024
INTER = 2048
LT_DIM = 768
TOKENS = 2048
SH = 2 * INTER

TM = 256
TN = 512
P = TOKENS + E * TM
NTP = P // TM
NK = INTER // TN
NTS = TOKENS // TM
NKS = SH // TN
TRASH = TOKENS

NC = 2
NS = 16
NW = NC * NS
TPW = TOKENS // NS
SPW = P // NW


def _silu(v):
    return v * jax.nn.sigmoid(v)


def _route_body(lt_ref, emb_ref, idx_ref):
    emb = emb_ref[...]
    enorm = jnp.sqrt(jnp.sum(emb * emb, axis=-1, keepdims=True))
    emb = emb / jnp.maximum(enorm, 1e-12)
    lt = lt_ref[...]
    tnorm = jnp.sqrt(jnp.sum(lt * lt, axis=-1, keepdims=True))
    lt = lt / jnp.maximum(tnorm, 1e-12)
    sims = lax.dot_general(lt, emb, (((1,), (1,)), ((), ())),
                           preferred_element_type=jnp.float32)
    idx_ref[...] = jnp.argmax(sims, axis=-1).astype(jnp.int32)


def _route(language_token, routing_embeddings):
    return pl.pallas_call(
        _route_body,
        grid=(1,),
        in_specs=[
            pl.BlockSpec((TOKENS, LT_DIM), lambda i: (0, 0)),
            pl.BlockSpec((E, LT_DIM), lambda i: (0, 0)),
        ],
        out_specs=pl.BlockSpec((TOKENS,), lambda i: (0,)),
        out_shape=jax.ShapeDtypeStruct((TOKENS,), jnp.int32),
    )(language_token, routing_embeddings)


PPW = P // NW
NCHUNK = TOKENS // 16


def _sort_body(idx_hbm, perm_hbm, texp_hbm, tvalid_hbm, stats_hbm,
               idx_v, chunk_v, meta_v, sem):
    cid = lax.axis_index("c")
    sid = lax.axis_index("s")
    w = sid * NC + cid
    base = w * PPW
    pltpu.sync_copy(idx_hbm, idx_v)
    lanes = lax.iota(jnp.int32, 16)

    def hist_step(i, counts):
        vec = idx_v[pl.ds(i * 16, 16)]
        for e in range(E):
            m = (vec == e).astype(jnp.int32)
            counts = counts + jnp.where(lanes == e, m * 0 + jnp.full(
                (16,), jnp.sum(m), jnp.int32), jnp.zeros((16,), jnp.int32))
        return counts
    totals = lax.fori_loop(0, NCHUNK, hist_step, jnp.zeros((16,), jnp.int32))

    padded = jnp.bitwise_and(totals + (TM - 1), -TM)
    incl = plsc.cumsum(padded)
    seg = incl - padded
    total_padded = jnp.max(incl)

    for i in range(PPW // 16):
        chunk_v[pl.ds(i * 16, 16)] = jnp.full((16,), TRASH, jnp.int32)
    lo = jnp.full((16,), base, jnp.int32)
    hi = jnp.full((16,), base + PPW, jnp.int32)

    def slot_step(i, cur):
        vec = idx_v[pl.ds(i * 16, 16)]
        svec = jnp.zeros((16,), jnp.int32)
        ncur = []
        for e in range(E):
            m = vec == e
            pos = plsc.cumsum(m.astype(jnp.int32))
            cvec = jnp.full((16,), cur[e], jnp.int32)
            svec = jnp.where(m, cvec + pos - 1, svec)
            ncur.append(cur[e] + pos[15])
        ids = lanes + jnp.full((16,), i * 16, jnp.int32)
        keep = jnp.logical_and(svec >= lo, svec < hi)
        plsc.store_scatter(chunk_v, [svec - lo], ids, mask=keep)
        return tuple(ncur)
    lax.fori_loop(0, NCHUNK, slot_step,
                  tuple(seg[e] for e in range(E)), unroll=2)

    pltpu.sync_copy(chunk_v, perm_hbm.at[pl.ds(base, PPW)])

    @pl.when(w == 0)
    def _meta():
        tp = jnp.full((16,), total_padded, jnp.int32)
        tile_starts = lax.iota(jnp.int32, 16) * TM
        valid = (tile_starts < tp).astype(jnp.int32)
        eff = jnp.minimum(tile_starts, tp - TM)
        acc = jnp.full((16,), -1, jnp.int32)
        for e in range(E):
            acc = acc + (eff >= jnp.full((16,), seg[e], jnp.int32)
                         ).astype(jnp.int32)
        meta_v[...] = acc
        pltpu.sync_copy(meta_v, texp_hbm)
        meta_v[...] = valid
        pltpu.sync_copy(meta_v, tvalid_hbm)
        meta_v[...] = tp
        pltpu.sync_copy(meta_v, stats_hbm)


def _sort(idx):
    mesh = plsc.VectorSubcoreMesh(core_axis_name="c", subcore_axis_name="s",
                                  num_cores=NC, num_subcores=NS)
    return pl.kernel(
        _sort_body,
        out_type=(
            jax.ShapeDtypeStruct((P,), jnp.int32),
            jax.ShapeDtypeStruct((16,), jnp.int32),
            jax.ShapeDtypeStruct((16,), jnp.int32),
            jax.ShapeDtypeStruct((16,), jnp.int32),
        ),
        mesh=mesh,
        compiler_params=pltpu.CompilerParams(needs_layout_passes=False),
        scratch_types=[
            pltpu.VMEM((TOKENS,), jnp.int32),
            pltpu.VMEM((PPW,), jnp.int32),
            pltpu.VMEM((16,), jnp.int32),
            pltpu.SemaphoreType.DMA,
        ],
    )(idx)


CH = 32
NCH = SPW // CH
NBUF = 3


def _gather_body(perm_hbm, x_hbm, stats_hbm, xs_hbm,
                 pidx_v, stat_v, ra_v, rb_v, rc_v, gsem, wsem):
    cid = lax.axis_index("c")
    sid = lax.axis_index("s")
    w = sid * NC + cid
    base = w * SPW
    pltpu.sync_copy(stats_hbm, stat_v)
    tp = stat_v[...][0]

    @pl.when(base < tp)
    def _():
        pltpu.sync_copy(perm_hbm.at[pl.ds(base, SPW)], pidx_v)
        for i in range(SPW // 16):
            pidx_v[pl.ds(i * 16, 16)] = jnp.minimum(
                pidx_v[pl.ds(i * 16, 16)], TOKENS - 1)
        bufs = (ra_v, rb_v, rc_v)
        gd = [None] * NCH
        wd = [None] * NCH
        for c in range(min(NBUF, NCH)):
            gd[c] = pltpu.async_copy(
                x_hbm.at[pidx_v.at[pl.ds(c * CH, CH)]], bufs[c % NBUF], gsem)
        for c in range(NCH):
            gd[c].wait()
            wd[c] = pltpu.async_copy(
                bufs[c % NBUF], xs_hbm.at[pl.ds(base + c * CH, CH)], wsem)
            if c + NBUF < NCH:
                wd[c].wait()
                gd[c + NBUF] = pltpu.async_copy(
                    x_hbm.at[pidx_v.at[pl.ds((c + NBUF) * CH, CH)]],
                    bufs[c % NBUF], gsem)
        for c in range(max(0, NCH - NBUF), NCH):
            wd[c].wait()


def _gather(perm, x, stats):
    mesh = plsc.VectorSubcoreMesh(core_axis_name="c", subcore_axis_name="s",
                                  num_cores=NC, num_subcores=NS)
    return pl.kernel(
        _gather_body,
        out_type=jax.ShapeDtypeStruct((P, DIM), jnp.float32),
        mesh=mesh,
        compiler_params=pltpu.CompilerParams(needs_layout_passes=False),
        scratch_types=[
            pltpu.VMEM((SPW,), jnp.int32),
            pltpu.VMEM((16,), jnp.int32),
            pltpu.VMEM((CH, DIM), jnp.float32),
            pltpu.VMEM((CH, DIM), jnp.float32),
            pltpu.VMEM((CH, DIM), jnp.float32),
            pltpu.SemaphoreType.DMA,
            pltpu.SemaphoreType.DMA,
        ],
    )(perm, x, stats)


def _gmm_body(texp_ref, tvalid_ref, xs_ref, ew_ref,
              w1_ref, b1_ref, w3_ref, b3_ref, w2_ref, b2_ref, out_ref):
    t = pl.program_id(0)
    k = pl.program_id(1)

    @pl.when(k == 0)
    def _init():
        out_ref[...] = jnp.zeros_like(out_ref)

    @pl.when(tvalid_ref[t] == 1)
    def _compute():
        scale = ew_ref[texp_ref[t]]
        x = xs_ref[...]
        a = lax.dot_general(x, w1_ref[0], (((1,), (0,)), ((), ())),
                            preferred_element_type=jnp.float32) + b1_ref[0, 0]
        g = lax.dot_general(x, w3_ref[0], (((1,), (0,)), ((), ())),
                            preferred_element_type=jnp.float32) + b3_ref[0, 0]
        h = _silu(a) * g
        o = lax.dot_general(h, w2_ref[0], (((1,), (0,)), ((), ())),
                            preferred_element_type=jnp.float32)

        @pl.when(k == 0)
        def _wb():
            out_ref[...] += scale * (o + b2_ref[0, 0])

        @pl.when(k != 0)
        def _nb():
            out_ref[...] += scale * o


def _gmm(texp, tvalid, xs, expert_weights, w1, b1, w3, b3, w2, b2):
    grid_spec = pltpu.PrefetchScalarGridSpec(
        num_scalar_prefetch=2,
        grid=(NTP, NK),
        in_specs=[
            pl.BlockSpec((TM, DIM), lambda t, k, te, tv: (t, 0)),
            pl.BlockSpec(memory_space=pltpu.SMEM),
            pl.BlockSpec((1, DIM, TN), lambda t, k, te, tv: (te[t], 0, k)),
            pl.BlockSpec((1, 1, TN), lambda t, k, te, tv: (te[t], 0, k)),
            pl.BlockSpec((1, DIM, TN), lambda t, k, te, tv: (te[t], 0, k)),
            pl.BlockSpec((1, 1, TN), lambda t, k, te, tv: (te[t], 0, k)),
            pl.BlockSpec((1, TN, DIM), lambda t, k, te, tv: (te[t], k, 0)),
            pl.BlockSpec((1, 1, DIM), lambda t, k, te, tv: (te[t], 0, 0)),
        ],
        out_specs=pl.BlockSpec((TM, DIM), lambda t, k, te, tv: (t, 0)),
    )
    return pl.pallas_call(
        _gmm_body,
        grid_spec=grid_spec,
        out_shape=jax.ShapeDtypeStruct((P, DIM), jnp.float32),
    )(texp, tvalid, xs, expert_weights,
      w1, b1.reshape(E, 1, INTER), w3, b3.reshape(E, 1, INTER),
      w2, b2.reshape(E, 1, DIM))


def _scatter_body(ys_hbm, perm_hbm, stats_hbm, yb_hbm,
                  i0_v, i1_v, i2_v, i3_v, stat_v, ra_v, rb_v, rc_v,
                  lsem, ssem):
    cid = lax.axis_index("c")
    sid = lax.axis_index("s")
    w = sid * NC + cid
    base = w * SPW
    pltpu.sync_copy(stats_hbm, stat_v)
    tp = stat_v[...][0]

    @pl.when(base < tp)
    def _():
        idxs = (i0_v, i1_v, i2_v, i3_v)
        for c in range(NCH):
            pltpu.sync_copy(perm_hbm.at[pl.ds(base + c * CH, CH)], idxs[c])
        bufs = (ra_v, rb_v, rc_v)
        ld = [None] * NCH
        sd = [None] * NCH
        for c in range(min(NBUF, NCH)):
            ld[c] = pltpu.async_copy(
                ys_hbm.at[pl.ds(base + c * CH, CH)], bufs[c % NBUF], lsem)
        for c in range(NCH):
            ld[c].wait()
            sd[c] = pltpu.async_copy(bufs[c % NBUF], yb_hbm.at[idxs[c]], ssem)
            if c + NBUF < NCH:
                sd[c].wait()
                ld[c + NBUF] = pltpu.async_copy(
                    ys_hbm.at[pl.ds(base + (c + NBUF) * CH, CH)],
                    bufs[c % NBUF], lsem)
        for c in range(max(0, NCH - NBUF), NCH):
            sd[c].wait()


def _scatter(y_sorted, perm, stats):
    mesh = plsc.VectorSubcoreMesh(core_axis_name="c", subcore_axis_name="s",
                                  num_cores=NC, num_subcores=NS)
    return pl.kernel(
        _scatter_body,
        out_type=jax.ShapeDtypeStruct((TOKENS + 8, DIM), jnp.float32),
        mesh=mesh,
        compiler_params=pltpu.CompilerParams(needs_layout_passes=False),
        scratch_types=[
            pltpu.VMEM((CH,), jnp.int32),
            pltpu.VMEM((CH,), jnp.int32),
            pltpu.VMEM((CH,), jnp.int32),
            pltpu.VMEM((CH,), jnp.int32),
            pltpu.VMEM((16,), jnp.int32),
            pltpu.VMEM((CH, DIM), jnp.float32),
            pltpu.VMEM((CH, DIM), jnp.float32),
            pltpu.VMEM((CH, DIM), jnp.float32),
            pltpu.SemaphoreType.DMA,
            pltpu.SemaphoreType.DMA,
        ],
    )(y_sorted, perm, stats)


def _shared_body(x_ref, y_ref, ws1_ref, bs1_ref, ws2_ref, bs2_ref, out_ref):
    k = pl.program_id(1)

    @pl.when(k == 0)
    def _init():
        out_ref[...] = y_ref[...] + bs2_ref[...]

    x = x_ref[...]
    a = lax.dot_general(x, ws1_ref[...], (((1,), (0,)), ((), ())),
                        preferred_element_type=jnp.float32) + bs1_ref[...]
    out_ref[...] += lax.dot_general(_silu(a), ws2_ref[...],
                                    (((1,), (0,)), ((), ())),
                                    preferred_element_type=jnp.float32)


def _shared(x, ybuf, ws1, bs1, ws2, bs2):
    return pl.pallas_call(
        _shared_body,
        grid=(NTS, NKS),
        in_specs=[
            pl.BlockSpec((TM, DIM), lambda t, k: (t, 0)),
            pl.BlockSpec((TM, DIM), lambda t, k: (t, 0)),
            pl.BlockSpec((DIM, TN), lambda t, k: (0, k)),
            pl.BlockSpec((1, TN), lambda t, k: (0, k)),
            pl.BlockSpec((TN, DIM), lambda t, k: (k, 0)),
            pl.BlockSpec((1, DIM), lambda t, k: (0, 0)),
        ],
        out_specs=pl.BlockSpec((TM, DIM), lambda t, k: (t, 0)),
        out_shape=jax.ShapeDtypeStruct((TOKENS, DIM), jnp.float32),
    )(x, ybuf, ws1, bs1.reshape(1, SH), ws2, bs2.reshape(1, DIM))


def kernel(x, language_token, routing_embeddings, expert_weights,
           w1, b1, w2, b2, w3, b3, ws1, bs1, ws2, bs2):
    idx = _route(language_token, routing_embeddings)
    perm, texp, tvalid, stats = _sort(idx)
    xs = _gather(perm, x, stats)
    ys = _gmm(texp, tvalid, xs, expert_weights, w1, b1, w3, b3, w2, b2)
    ybuf = _scatter(ys, perm, stats)
    return _shared(x, ybuf, ws1, bs1, ws2, bs2)


# speedup vs baseline: 2.2517x; 1.1322x over previous
import functools

import jax
import jax.numpy as jnp
from jax import lax
from jax.experimental import pallas as pl
from jax.experimental.pallas import tpu as pltpu
from jax.experimental.pallas import tpu_sc as plsc

E = 8
DIM = 1024
INTER = 2048
LT_DIM = 768
TOKENS = 2048
SH = 2 * INTER

TM = 256
TN = 1024
P = TOKENS + E * TM
NTP = P // TM
NK = INTER // TN
NTS = TOKENS // TM
NKS = SH // TN
TRASH = TOKENS

NC = 2
NS = 16
NW = NC * NS
TPW = TOKENS // NS
SPW = P // NW


def _silu(v):
    return v * jax.nn.sigmoid(v)


def _route_body(lt_ref, emb_ref, idx_ref):
    emb = emb_ref[...]
    enorm = jnp.sqrt(jnp.sum(emb * emb, axis=-1, keepdims=True))
    emb = emb / jnp.maximum(enorm, 1e-12)
    lt = lt_ref[...]
    tnorm = jnp.sqrt(jnp.sum(lt * lt, axis=-1, keepdims=True))
    lt = lt / jnp.maximum(tnorm, 1e-12)
    sims = lax.dot_general(lt, emb, (((1,), (1,)), ((), ())),
                           preferred_element_type=jnp.float32)
    idx_ref[...] = jnp.argmax(sims, axis=-1).astype(jnp.int32)


def _route(language_token, routing_embeddings):
    return pl.pallas_call(
        _route_body,
        grid=(1,),
        in_specs=[
            pl.BlockSpec((TOKENS, LT_DIM), lambda i: (0, 0)),
            pl.BlockSpec((E, LT_DIM), lambda i: (0, 0)),
        ],
        out_specs=pl.BlockSpec((TOKENS,), lambda i: (0,)),
        out_shape=jax.ShapeDtypeStruct((TOKENS,), jnp.int32),
    )(language_token, routing_embeddings)


PPW = P // NW
NCHUNK = TOKENS // 16


def _sort_body(idx_hbm, perm_hbm, texp_hbm, tvalid_hbm, stats_hbm,
               idx_v, chunk_v, meta_v, sem):
    cid = lax.axis_index("c")
    sid = lax.axis_index("s")
    w = sid * NC + cid
    base = w * PPW
    pltpu.sync_copy(idx_hbm, idx_v)
    lanes = lax.iota(jnp.int32, 16)

    def hist_step(i, counts):
        vec = idx_v[pl.ds(i * 16, 16)]
        for e in range(E):
            m = (vec == e).astype(jnp.int32)
            counts = counts + jnp.where(lanes == e, m * 0 + jnp.full(
                (16,), jnp.sum(m), jnp.int32), jnp.zeros((16,), jnp.int32))
        return counts
    totals = lax.fori_loop(0, NCHUNK, hist_step, jnp.zeros((16,), jnp.int32))

    padded = jnp.bitwise_and(totals + (TM - 1), -TM)
    incl = plsc.cumsum(padded)
    seg = incl - padded
    total_padded = jnp.max(incl)

    for i in range(PPW // 16):
        chunk_v[pl.ds(i * 16, 16)] = jnp.full((16,), TRASH, jnp.int32)
    lo = jnp.full((16,), base, jnp.int32)
    hi = jnp.full((16,), base + PPW, jnp.int32)

    def slot_step(i, cur):
        vec = idx_v[pl.ds(i * 16, 16)]
        svec = jnp.zeros((16,), jnp.int32)
        ncur = []
        for e in range(E):
            m = vec == e
            pos = plsc.cumsum(m.astype(jnp.int32))
            cvec = jnp.full((16,), cur[e], jnp.int32)
            svec = jnp.where(m, cvec + pos - 1, svec)
            ncur.append(cur[e] + pos[15])
        ids = lanes + jnp.full((16,), i * 16, jnp.int32)
        keep = jnp.logical_and(svec >= lo, svec < hi)
        plsc.store_scatter(chunk_v, [svec - lo], ids, mask=keep)
        return tuple(ncur)
    lax.fori_loop(0, NCHUNK, slot_step,
                  tuple(seg[e] for e in range(E)), unroll=2)

    pltpu.sync_copy(chunk_v, perm_hbm.at[pl.ds(base, PPW)])

    @pl.when(w == 0)
    def _meta():
        tp = jnp.full((16,), total_padded, jnp.int32)
        tile_starts = lax.iota(jnp.int32, 16) * TM
        valid = (tile_starts < tp).astype(jnp.int32)
        eff = jnp.minimum(tile_starts, tp - TM)
        acc = jnp.full((16,), -1, jnp.int32)
        for e in range(E):
            acc = acc + (eff >= jnp.full((16,), seg[e], jnp.int32)
                         ).astype(jnp.int32)
        meta_v[...] = acc
        pltpu.sync_copy(meta_v, texp_hbm)
        meta_v[...] = valid
        pltpu.sync_copy(meta_v, tvalid_hbm)
        meta_v[...] = tp
        pltpu.sync_copy(meta_v, stats_hbm)


def _sort(idx):
    mesh = plsc.VectorSubcoreMesh(core_axis_name="c", subcore_axis_name="s",
                                  num_cores=NC, num_subcores=NS)
    return pl.kernel(
        _sort_body,
        out_type=(
            jax.ShapeDtypeStruct((P,), jnp.int32),
            jax.ShapeDtypeStruct((16,), jnp.int32),
            jax.ShapeDtypeStruct((16,), jnp.int32),
            jax.ShapeDtypeStruct((16,), jnp.int32),
        ),
        mesh=mesh,
        compiler_params=pltpu.CompilerParams(needs_layout_passes=False),
        scratch_types=[
            pltpu.VMEM((TOKENS,), jnp.int32),
            pltpu.VMEM((PPW,), jnp.int32),
            pltpu.VMEM((16,), jnp.int32),
            pltpu.SemaphoreType.DMA,
        ],
    )(idx)


CH = 32
NCH = SPW // CH
NBUF = 3


def _gather_body(perm_hbm, x_hbm, stats_hbm, xs_hbm,
                 pidx_v, stat_v, ra_v, rb_v, rc_v, gsem, wsem):
    cid = lax.axis_index("c")
    sid = lax.axis_index("s")
    w = sid * NC + cid
    base = w * SPW
    pltpu.sync_copy(stats_hbm, stat_v)
    tp = stat_v[...][0]

    @pl.when(base < tp)
    def _():
        pltpu.sync_copy(perm_hbm.at[pl.ds(base, SPW)], pidx_v)
        for i in range(SPW // 16):
            pidx_v[pl.ds(i * 16, 16)] = jnp.minimum(
                pidx_v[pl.ds(i * 16, 16)], TOKENS - 1)
        bufs = (ra_v, rb_v, rc_v)
        gd = [None] * NCH
        wd = [None] * NCH
        for c in range(min(NBUF, NCH)):
            gd[c] = pltpu.async_copy(
                x_hbm.at[pidx_v.at[pl.ds(c * CH, CH)]], bufs[c % NBUF], gsem)
        for c in range(NCH):
            gd[c].wait()
            wd[c] = pltpu.async_copy(
                bufs[c % NBUF], xs_hbm.at[pl.ds(base + c * CH, CH)], wsem)
            if c + NBUF < NCH:
                wd[c].wait()
                gd[c + NBUF] = pltpu.async_copy(
                    x_hbm.at[pidx_v.at[pl.ds((c + NBUF) * CH, CH)]],
                    bufs[c % NBUF], gsem)
        for c in range(max(0, NCH - NBUF), NCH):
            wd[c].wait()


def _gather(perm, x, stats):
    mesh = plsc.VectorSubcoreMesh(core_axis_name="c", subcore_axis_name="s",
                                  num_cores=NC, num_subcores=NS)
    return pl.kernel(
        _gather_body,
        out_type=jax.ShapeDtypeStruct((P, DIM), jnp.float32),
        mesh=mesh,
        compiler_params=pltpu.CompilerParams(needs_layout_passes=False),
        scratch_types=[
            pltpu.VMEM((SPW,), jnp.int32),
            pltpu.VMEM((16,), jnp.int32),
            pltpu.VMEM((CH, DIM), jnp.float32),
            pltpu.VMEM((CH, DIM), jnp.float32),
            pltpu.VMEM((CH, DIM), jnp.float32),
            pltpu.SemaphoreType.DMA,
            pltpu.SemaphoreType.DMA,
        ],
    )(perm, x, stats)


def _gmm_body(texp_ref, tvalid_ref, xs_ref, ew_ref,
              w1_ref, b1_ref, w3_ref, b3_ref, w2_ref, b2_ref, out_ref):
    t = pl.program_id(0)
    k = pl.program_id(1)

    @pl.when(k == 0)
    def _init():
        out_ref[...] = jnp.zeros_like(out_ref)

    @pl.when(tvalid_ref[t] == 1)
    def _compute():
        x = xs_ref[...]
        a = lax.dot_general(x, w1_ref[0], (((1,), (0,)), ((), ())),
                            preferred_element_type=jnp.float32) + b1_ref[0, 0]
        g = lax.dot_general(x, w3_ref[0], (((1,), (0,)), ((), ())),
                            preferred_element_type=jnp.float32) + b3_ref[0, 0]
        h = _silu(a) * g
        o = lax.dot_general(h, w2_ref[0], (((1,), (0,)), ((), ())),
                            preferred_element_type=jnp.float32)

        @pl.when(k == 0)
        def _wb():
            out_ref[...] += o + b2_ref[0, 0]

        @pl.when(k != 0)
        def _nb():
            out_ref[...] += o

        @pl.when(k == NK - 1)
        def _sc():
            out_ref[...] *= ew_ref[texp_ref[t]]


def _gmm(texp, tvalid, xs, expert_weights, w1, b1, w3, b3, w2, b2):
    grid_spec = pltpu.PrefetchScalarGridSpec(
        num_scalar_prefetch=2,
        grid=(NTP, NK),
        in_specs=[
            pl.BlockSpec((TM, DIM), lambda t, k, te, tv: (t, 0)),
            pl.BlockSpec(memory_space=pltpu.SMEM),
            pl.BlockSpec((1, DIM, TN), lambda t, k, te, tv: (te[t], 0, k)),
            pl.BlockSpec((1, 1, TN), lambda t, k, te, tv: (te[t], 0, k)),
            pl.BlockSpec((1, DIM, TN), lambda t, k, te, tv: (te[t], 0, k)),
            pl.BlockSpec((1, 1, TN), lambda t, k, te, tv: (te[t], 0, k)),
            pl.BlockSpec((1, TN, DIM), lambda t, k, te, tv: (te[t], k, 0)),
            pl.BlockSpec((1, 1, DIM), lambda t, k, te, tv: (te[t], 0, 0)),
        ],
        out_specs=pl.BlockSpec((TM, DIM), lambda t, k, te, tv: (t, 0)),
    )
    return pl.pallas_call(
        _gmm_body,
        grid_spec=grid_spec,
        out_shape=jax.ShapeDtypeStruct((P, DIM), jnp.float32),
    )(texp, tvalid, xs, expert_weights,
      w1, b1.reshape(E, 1, INTER), w3, b3.reshape(E, 1, INTER),
      w2, b2.reshape(E, 1, DIM))


def _scatter_body(ys_hbm, perm_hbm, stats_hbm, yb_hbm,
                  i0_v, i1_v, i2_v, i3_v, stat_v, ra_v, rb_v, rc_v,
                  lsem, ssem):
    cid = lax.axis_index("c")
    sid = lax.axis_index("s")
    w = sid * NC + cid
    base = w * SPW
    pltpu.sync_copy(stats_hbm, stat_v)
    tp = stat_v[...][0]

    @pl.when(base < tp)
    def _():
        idxs = (i0_v, i1_v, i2_v, i3_v)
        for c in range(NCH):
            pltpu.sync_copy(perm_hbm.at[pl.ds(base + c * CH, CH)], idxs[c])
        bufs = (ra_v, rb_v, rc_v)
        ld = [None] * NCH
        sd = [None] * NCH
        for c in range(min(NBUF, NCH)):
            ld[c] = pltpu.async_copy(
                ys_hbm.at[pl.ds(base + c * CH, CH)], bufs[c % NBUF], lsem)
        for c in range(NCH):
            ld[c].wait()
            sd[c] = pltpu.async_copy(bufs[c % NBUF], yb_hbm.at[idxs[c]], ssem)
            if c + NBUF < NCH:
                sd[c].wait()
                ld[c + NBUF] = pltpu.async_copy(
                    ys_hbm.at[pl.ds(base + (c + NBUF) * CH, CH)],
                    bufs[c % NBUF], lsem)
        for c in range(max(0, NCH - NBUF), NCH):
            sd[c].wait()


def _scatter(y_sorted, perm, stats):
    mesh = plsc.VectorSubcoreMesh(core_axis_name="c", subcore_axis_name="s",
                                  num_cores=NC, num_subcores=NS)
    return pl.kernel(
        _scatter_body,
        out_type=jax.ShapeDtypeStruct((TOKENS + 8, DIM), jnp.float32),
        mesh=mesh,
        compiler_params=pltpu.CompilerParams(needs_layout_passes=False),
        scratch_types=[
            pltpu.VMEM((CH,), jnp.int32),
            pltpu.VMEM((CH,), jnp.int32),
            pltpu.VMEM((CH,), jnp.int32),
            pltpu.VMEM((CH,), jnp.int32),
            pltpu.VMEM((16,), jnp.int32),
            pltpu.VMEM((CH, DIM), jnp.float32),
            pltpu.VMEM((CH, DIM), jnp.float32),
            pltpu.VMEM((CH, DIM), jnp.float32),
            pltpu.SemaphoreType.DMA,
            pltpu.SemaphoreType.DMA,
        ],
    )(y_sorted, perm, stats)


def _shared_body(x_ref, ws1_ref, bs1_ref, ws2_ref, bs2_ref, out_ref):
    k = pl.program_id(1)

    @pl.when(k == 0)
    def _init():
        out_ref[...] = jnp.broadcast_to(bs2_ref[...], out_ref.shape)

    a = lax.dot_general(x_ref[...], ws1_ref[...], (((1,), (0,)), ((), ())),
                        preferred_element_type=jnp.float32) + bs1_ref[...]
    h = _silu(a).astype(jnp.bfloat16)
    out_ref[...] += lax.dot_general(h, ws2_ref[...],
                                    (((1,), (0,)), ((), ())),
                                    preferred_element_type=jnp.float32)


def _shared(x, ws1, bs1, ws2, bs2):
    return pl.pallas_call(
        _shared_body,
        grid=(NTS, NKS),
        in_specs=[
            pl.BlockSpec((TM, DIM), lambda t, k: (t, 0)),
            pl.BlockSpec((DIM, TN), lambda t, k: (0, k)),
            pl.BlockSpec((1, TN), lambda t, k: (0, k)),
            pl.BlockSpec((TN, DIM), lambda t, k: (k, 0)),
            pl.BlockSpec((1, DIM), lambda t, k: (0, 0)),
        ],
        out_specs=pl.BlockSpec((TM, DIM), lambda t, k: (t, 0)),
        out_shape=jax.ShapeDtypeStruct((TOKENS, DIM), jnp.float32),
    )(x.astype(jnp.bfloat16), ws1.astype(jnp.bfloat16), bs1.reshape(1, SH),
      ws2.astype(jnp.bfloat16), bs2.reshape(1, DIM))


def _add_body(z_ref, y_ref, out_ref):
    out_ref[...] = z_ref[...] + y_ref[...]


def _final_add(z, ybuf):
    return pl.pallas_call(
        _add_body,
        grid=(NTS,),
        in_specs=[
            pl.BlockSpec((TM, DIM), lambda t: (t, 0)),
            pl.BlockSpec((TM, DIM), lambda t: (t, 0)),
        ],
        out_specs=pl.BlockSpec((TM, DIM), lambda t: (t, 0)),
        out_shape=jax.ShapeDtypeStruct((TOKENS, DIM), jnp.float32),
    )(z, ybuf)


def kernel(x, language_token, routing_embeddings, expert_weights,
           w1, b1, w2, b2, w3, b3, ws1, bs1, ws2, bs2):
    z = _shared(x, ws1, bs1, ws2, bs2)
    idx = _route(language_token, routing_embeddings)
    perm, texp, tvalid, stats = _sort(idx)
    xs = _gather(perm, x, stats)
    ys = _gmm(texp, tvalid, xs, expert_weights, w1, b1, w3, b3, w2, b2)
    ybuf = _scatter(ys, perm, stats)
    return _final_add(z, ybuf)


# speedup vs baseline: 2.2783x; 1.0118x over previous
import functools

import jax
import jax.numpy as jnp
from jax import lax
from jax.experimental import pallas as pl
from jax.experimental.pallas import tpu as pltpu
from jax.experimental.pallas import tpu_sc as plsc

E = 8
DIM = 1024
INTER = 2048
LT_DIM = 768
TOKENS = 2048
SH = 2 * INTER

TM = 256
TN = 1024
P = TOKENS + E * TM
NTP = P // TM
NK = INTER // TN
NTS = TOKENS // TM
NKS = SH // TN
TRASH = TOKENS

NC = 2
NS = 16
NW = NC * NS
TPW = TOKENS // NS
SPW = P // NW


def _silu(v):
    return v * jax.nn.sigmoid(v)


def _route_body(lt_ref, emb_ref, idx_ref):
    emb = emb_ref[...]
    enorm = jnp.sqrt(jnp.sum(emb * emb, axis=-1, keepdims=True))
    emb = emb / jnp.maximum(enorm, 1e-12)
    lt = lt_ref[...]
    tnorm = jnp.sqrt(jnp.sum(lt * lt, axis=-1, keepdims=True))
    lt = lt / jnp.maximum(tnorm, 1e-12)
    sims = lax.dot_general(lt, emb, (((1,), (1,)), ((), ())),
                           preferred_element_type=jnp.float32)
    idx_ref[...] = jnp.argmax(sims, axis=-1).astype(jnp.int32)


def _route(language_token, routing_embeddings):
    return pl.pallas_call(
        _route_body,
        grid=(1,),
        in_specs=[
            pl.BlockSpec((TOKENS, LT_DIM), lambda i: (0, 0)),
            pl.BlockSpec((E, LT_DIM), lambda i: (0, 0)),
        ],
        out_specs=pl.BlockSpec((TOKENS,), lambda i: (0,)),
        out_shape=jax.ShapeDtypeStruct((TOKENS,), jnp.int32),
    )(language_token, routing_embeddings)


PPW = P // NW
NCHUNK = TOKENS // 16


def _sort_body(idx_hbm, perm_hbm, texp_hbm, tvalid_hbm, stats_hbm,
               idx_v, chunk_v, meta_v, sem):
    cid = lax.axis_index("c")
    sid = lax.axis_index("s")
    w = sid * NC + cid
    base = w * PPW
    pltpu.sync_copy(idx_hbm, idx_v)
    lanes = lax.iota(jnp.int32, 16)

    def hist_step(i, counts):
        vec = idx_v[pl.ds(i * 16, 16)]
        for e in range(E):
            m = (vec == e).astype(jnp.int32)
            counts = counts + jnp.where(lanes == e, m * 0 + jnp.full(
                (16,), jnp.sum(m), jnp.int32), jnp.zeros((16,), jnp.int32))
        return counts
    totals = lax.fori_loop(0, NCHUNK, hist_step, jnp.zeros((16,), jnp.int32))

    padded = jnp.bitwise_and(totals + (TM - 1), -TM)
    incl = plsc.cumsum(padded)
    seg = incl - padded
    total_padded = jnp.max(incl)

    for i in range(PPW // 16):
        chunk_v[pl.ds(i * 16, 16)] = jnp.full((16,), TRASH, jnp.int32)
    lo = jnp.full((16,), base, jnp.int32)
    hi = jnp.full((16,), base + PPW, jnp.int32)

    def slot_step(i, cur):
        vec = idx_v[pl.ds(i * 16, 16)]
        svec = jnp.zeros((16,), jnp.int32)
        ncur = []
        for e in range(E):
            m = vec == e
            pos = plsc.cumsum(m.astype(jnp.int32))
            cvec = jnp.full((16,), cur[e], jnp.int32)
            svec = jnp.where(m, cvec + pos - 1, svec)
            ncur.append(cur[e] + pos[15])
        ids = lanes + jnp.full((16,), i * 16, jnp.int32)
        keep = jnp.logical_and(svec >= lo, svec < hi)
        plsc.store_scatter(chunk_v, [svec - lo], ids, mask=keep)
        return tuple(ncur)
    lax.fori_loop(0, NCHUNK, slot_step,
                  tuple(seg[e] for e in range(E)), unroll=2)

    pltpu.sync_copy(chunk_v, perm_hbm.at[pl.ds(base, PPW)])

    @pl.when(w == 0)
    def _meta():
        tp = jnp.full((16,), total_padded, jnp.int32)
        tile_starts = lax.iota(jnp.int32, 16) * TM
        valid = (tile_starts < tp).astype(jnp.int32)
        eff = jnp.minimum(tile_starts, tp - TM)
        acc = jnp.full((16,), -1, jnp.int32)
        for e in range(E):
            acc = acc + (eff >= jnp.full((16,), seg[e], jnp.int32)
                         ).astype(jnp.int32)
        meta_v[...] = acc
        pltpu.sync_copy(meta_v, texp_hbm)
        meta_v[...] = valid
        pltpu.sync_copy(meta_v, tvalid_hbm)
        meta_v[...] = tp
        pltpu.sync_copy(meta_v, stats_hbm)


def _sort(idx):
    mesh = plsc.VectorSubcoreMesh(core_axis_name="c", subcore_axis_name="s",
                                  num_cores=NC, num_subcores=NS)
    return pl.kernel(
        _sort_body,
        out_type=(
            jax.ShapeDtypeStruct((P,), jnp.int32),
            jax.ShapeDtypeStruct((16,), jnp.int32),
            jax.ShapeDtypeStruct((16,), jnp.int32),
            jax.ShapeDtypeStruct((16,), jnp.int32),
        ),
        mesh=mesh,
        compiler_params=pltpu.CompilerParams(needs_layout_passes=False),
        scratch_types=[
            pltpu.VMEM((TOKENS,), jnp.int32),
            pltpu.VMEM((PPW,), jnp.int32),
            pltpu.VMEM((16,), jnp.int32),
            pltpu.SemaphoreType.DMA,
        ],
    )(idx)


CH = 32
NCH = SPW // CH
NBUF = 3


def _gather_body(perm_hbm, x_hbm, stats_hbm, xs_hbm,
                 pidx_v, stat_v, ra_v, rb_v, rc_v, gsem, wsem):
    cid = lax.axis_index("c")
    sid = lax.axis_index("s")
    w = sid * NC + cid
    base = w * SPW
    pltpu.sync_copy(stats_hbm, stat_v)
    tp = stat_v[...][0]

    @pl.when(base < tp)
    def _():
        pltpu.sync_copy(perm_hbm.at[pl.ds(base, SPW)], pidx_v)
        for i in range(SPW // 16):
            pidx_v[pl.ds(i * 16, 16)] = jnp.minimum(
                pidx_v[pl.ds(i * 16, 16)], TOKENS - 1)
        bufs = (ra_v, rb_v, rc_v)
        gd = [None] * NCH
        wd = [None] * NCH
        for c in range(min(NBUF, NCH)):
            gd[c] = pltpu.async_copy(
                x_hbm.at[pidx_v.at[pl.ds(c * CH, CH)]], bufs[c % NBUF], gsem)
        for c in range(NCH):
            gd[c].wait()
            wd[c] = pltpu.async_copy(
                bufs[c % NBUF], xs_hbm.at[pl.ds(base + c * CH, CH)], wsem)
            if c + NBUF < NCH:
                wd[c].wait()
                gd[c + NBUF] = pltpu.async_copy(
                    x_hbm.at[pidx_v.at[pl.ds((c + NBUF) * CH, CH)]],
                    bufs[c % NBUF], gsem)
        for c in range(max(0, NCH - NBUF), NCH):
            wd[c].wait()


def _gather(perm, x, stats):
    mesh = plsc.VectorSubcoreMesh(core_axis_name="c", subcore_axis_name="s",
                                  num_cores=NC, num_subcores=NS)
    return pl.kernel(
        _gather_body,
        out_type=jax.ShapeDtypeStruct((P, DIM), jnp.float32),
        mesh=mesh,
        compiler_params=pltpu.CompilerParams(needs_layout_passes=False),
        scratch_types=[
            pltpu.VMEM((SPW,), jnp.int32),
            pltpu.VMEM((16,), jnp.int32),
            pltpu.VMEM((CH, DIM), jnp.float32),
            pltpu.VMEM((CH, DIM), jnp.float32),
            pltpu.VMEM((CH, DIM), jnp.float32),
            pltpu.SemaphoreType.DMA,
            pltpu.SemaphoreType.DMA,
        ],
    )(perm, x, stats)


def _gmm_body(texp_ref, tvalid_ref, xs_ref, ew_ref,
              w1_ref, b1_ref, w3_ref, b3_ref, w2_ref, b2_ref, out_ref):
    t = pl.program_id(0)
    k = pl.program_id(1)

    @pl.when(k == 0)
    def _init():
        out_ref[...] = jnp.zeros_like(out_ref)

    @pl.when(tvalid_ref[t] == 1)
    def _compute():
        x = xs_ref[...]
        a = lax.dot_general(x, w1_ref[0], (((1,), (0,)), ((), ())),
                            preferred_element_type=jnp.float32) + b1_ref[0, 0]
        g = lax.dot_general(x, w3_ref[0], (((1,), (0,)), ((), ())),
                            preferred_element_type=jnp.float32) + b3_ref[0, 0]
        h = _silu(a) * g
        o = lax.dot_general(h, w2_ref[0], (((1,), (0,)), ((), ())),
                            preferred_element_type=jnp.float32)

        @pl.when(k == 0)
        def _wb():
            out_ref[...] += o + b2_ref[0, 0]

        @pl.when(k != 0)
        def _nb():
            out_ref[...] += o

        @pl.when(k == NK - 1)
        def _sc():
            out_ref[...] *= ew_ref[texp_ref[t]]


def _gmm(texp, tvalid, xs, expert_weights, w1, b1, w3, b3, w2, b2):
    grid_spec = pltpu.PrefetchScalarGridSpec(
        num_scalar_prefetch=2,
        grid=(NTP, NK),
        in_specs=[
            pl.BlockSpec((TM, DIM), lambda t, k, te, tv: (t, 0)),
            pl.BlockSpec(memory_space=pltpu.SMEM),
            pl.BlockSpec((1, DIM, TN), lambda t, k, te, tv: (te[t], 0, k)),
            pl.BlockSpec((1, 1, TN), lambda t, k, te, tv: (te[t], 0, k)),
            pl.BlockSpec((1, DIM, TN), lambda t, k, te, tv: (te[t], 0, k)),
            pl.BlockSpec((1, 1, TN), lambda t, k, te, tv: (te[t], 0, k)),
            pl.BlockSpec((1, TN, DIM), lambda t, k, te, tv: (te[t], k, 0)),
            pl.BlockSpec((1, 1, DIM), lambda t, k, te, tv: (te[t], 0, 0)),
        ],
        out_specs=pl.BlockSpec((TM, DIM), lambda t, k, te, tv: (t, 0)),
    )
    return pl.pallas_call(
        _gmm_body,
        grid_spec=grid_spec,
        out_shape=jax.ShapeDtypeStruct((P, DIM), jnp.float32),
    )(texp, tvalid, xs, expert_weights,
      w1, b1.reshape(E, 1, INTER), w3, b3.reshape(E, 1, INTER),
      w2, b2.reshape(E, 1, DIM))


def _scatter_body(ys_hbm, perm_hbm, stats_hbm, yb_hbm,
                  i0_v, i1_v, i2_v, i3_v, stat_v, ra_v, rb_v, rc_v,
                  lsem, ssem):
    cid = lax.axis_index("c")
    sid = lax.axis_index("s")
    w = sid * NC + cid
    base = w * SPW
    pltpu.sync_copy(stats_hbm, stat_v)
    tp = stat_v[...][0]

    @pl.when(base < tp)
    def _():
        idxs = (i0_v, i1_v, i2_v, i3_v)
        for c in range(NCH):
            pltpu.sync_copy(perm_hbm.at[pl.ds(base + c * CH, CH)], idxs[c])
        bufs = (ra_v, rb_v, rc_v)
        ld = [None] * NCH
        sd = [None] * NCH
        for c in range(min(NBUF, NCH)):
            ld[c] = pltpu.async_copy(
                ys_hbm.at[pl.ds(base + c * CH, CH)], bufs[c % NBUF], lsem)
        for c in range(NCH):
            ld[c].wait()
            sd[c] = pltpu.async_copy(bufs[c % NBUF], yb_hbm.at[idxs[c]], ssem)
            if c + NBUF < NCH:
                sd[c].wait()
                ld[c + NBUF] = pltpu.async_copy(
                    ys_hbm.at[pl.ds(base + (c + NBUF) * CH, CH)],
                    bufs[c % NBUF], lsem)
        for c in range(max(0, NCH - NBUF), NCH):
            sd[c].wait()


def _scatter(y_sorted, perm, stats):
    mesh = plsc.VectorSubcoreMesh(core_axis_name="c", subcore_axis_name="s",
                                  num_cores=NC, num_subcores=NS)
    return pl.kernel(
        _scatter_body,
        out_type=jax.ShapeDtypeStruct((TOKENS + 8, DIM), jnp.float32),
        mesh=mesh,
        compiler_params=pltpu.CompilerParams(needs_layout_passes=False),
        scratch_types=[
            pltpu.VMEM((CH,), jnp.int32),
            pltpu.VMEM((CH,), jnp.int32),
            pltpu.VMEM((CH,), jnp.int32),
            pltpu.VMEM((CH,), jnp.int32),
            pltpu.VMEM((16,), jnp.int32),
            pltpu.VMEM((CH, DIM), jnp.float32),
            pltpu.VMEM((CH, DIM), jnp.float32),
            pltpu.VMEM((CH, DIM), jnp.float32),
            pltpu.SemaphoreType.DMA,
            pltpu.SemaphoreType.DMA,
        ],
    )(y_sorted, perm, stats)


def _shared_body(x_ref, y_ref, ws1_ref, bs1_ref, ws2_ref, bs2_ref, out_ref):
    k = pl.program_id(1)

    @pl.when(k == 0)
    def _init():
        out_ref[...] = y_ref[...] + bs2_ref[...]

    a = lax.dot_general(x_ref[...], ws1_ref[...], (((1,), (0,)), ((), ())),
                        preferred_element_type=jnp.float32) + bs1_ref[...]
    h = _silu(a).astype(jnp.bfloat16)
    out_ref[...] += lax.dot_general(h, ws2_ref[...],
                                    (((1,), (0,)), ((), ())),
                                    preferred_element_type=jnp.float32)


def _shared(x, ybuf, ws1, bs1, ws2, bs2):
    return pl.pallas_call(
        _shared_body,
        grid=(NTS, NKS),
        in_specs=[
            pl.BlockSpec((TM, DIM), lambda t, k: (t, 0)),
            pl.BlockSpec((TM, DIM), lambda t, k: (t, 0)),
            pl.BlockSpec((DIM, TN), lambda t, k: (0, k)),
            pl.BlockSpec((1, TN), lambda t, k: (0, k)),
            pl.BlockSpec((TN, DIM), lambda t, k: (k, 0)),
            pl.BlockSpec((1, DIM), lambda t, k: (0, 0)),
        ],
        out_specs=pl.BlockSpec((TM, DIM), lambda t, k: (t, 0)),
        out_shape=jax.ShapeDtypeStruct((TOKENS, DIM), jnp.float32),
    )(x.astype(jnp.bfloat16), ybuf, ws1.astype(jnp.bfloat16),
      bs1.reshape(1, SH), ws2.astype(jnp.bfloat16), bs2.reshape(1, DIM))


def _add_body(z_ref, y_ref, out_ref):
    out_ref[...] = z_ref[...] + y_ref[...]


def _final_add(z, ybuf):
    return pl.pallas_call(
        _add_body,
        grid=(NTS,),
        in_specs=[
            pl.BlockSpec((TM, DIM), lambda t: (t, 0)),
            pl.BlockSpec((TM, DIM), lambda t: (t, 0)),
        ],
        out_specs=pl.BlockSpec((TM, DIM), lambda t: (t, 0)),
        out_shape=jax.ShapeDtypeStruct((TOKENS, DIM), jnp.float32),
    )(z, ybuf)


def kernel(x, language_token, routing_embeddings, expert_weights,
           w1, b1, w2, b2, w3, b3, ws1, bs1, ws2, bs2):
    idx = _route(language_token, routing_embeddings)
    perm, texp, tvalid, stats = _sort(idx)
    xs = _gather(perm, x, stats)
    ys = _gmm(texp, tvalid, xs, expert_weights, w1, b1, w3, b3, w2, b2)
    ybuf = _scatter(ys, perm, stats)
    return _shared(x, ybuf, ws1, bs1, ws2, bs2)


# speedup vs baseline: 2.4303x; 1.0667x over previous
import functools

import jax
import jax.numpy as jnp
from jax import lax
from jax.experimental import pallas as pl
from jax.experimental.pallas import tpu as pltpu
from jax.experimental.pallas import tpu_sc as plsc

E = 8
DIM = 1024
INTER = 2048
LT_DIM = 768
TOKENS = 2048
SH = 2 * INTER

TM = 256
TN = 1024
P = TOKENS + E * TM
NTP = P // TM
NK = INTER // TN
NTS = TOKENS // TM
NKS = SH // TN
TRASH = TOKENS

NC = 2
NS = 16
NW = NC * NS
TPW = TOKENS // NS
SPW = P // NW


def _silu(v):
    return v * jax.nn.sigmoid(v)


def _route_body(lt_ref, emb_ref, idx_ref):
    emb = emb_ref[...]
    enorm = jnp.sqrt(jnp.sum(emb * emb, axis=-1, keepdims=True))
    emb = emb / jnp.maximum(enorm, 1e-12)
    lt = lt_ref[...]
    tnorm = jnp.sqrt(jnp.sum(lt * lt, axis=-1, keepdims=True))
    lt = lt / jnp.maximum(tnorm, 1e-12)
    sims = lax.dot_general(lt, emb, (((1,), (1,)), ((), ())),
                           preferred_element_type=jnp.float32)
    idx_ref[...] = jnp.argmax(sims, axis=-1).astype(jnp.int32)


def _route(language_token, routing_embeddings):
    return pl.pallas_call(
        _route_body,
        grid=(1,),
        in_specs=[
            pl.BlockSpec((TOKENS, LT_DIM), lambda i: (0, 0)),
            pl.BlockSpec((E, LT_DIM), lambda i: (0, 0)),
        ],
        out_specs=pl.BlockSpec((TOKENS,), lambda i: (0,)),
        out_shape=jax.ShapeDtypeStruct((TOKENS,), jnp.int32),
    )(language_token, routing_embeddings)


PPW = P // NW
NCHUNK = TOKENS // 16


def _sort_body(idx_hbm, perm_hbm, texp_hbm, tvalid_hbm, stats_hbm,
               idx_v, chunk_v, meta_v, sem):
    cid = lax.axis_index("c")
    sid = lax.axis_index("s")
    w = sid * NC + cid
    base = w * PPW
    pltpu.sync_copy(idx_hbm, idx_v)
    lanes = lax.iota(jnp.int32, 16)

    def hist_step(i, counts):
        vec = idx_v[pl.ds(i * 16, 16)]
        for e in range(E):
            m = (vec == e).astype(jnp.int32)
            counts = counts + jnp.where(lanes == e, m * 0 + jnp.full(
                (16,), jnp.sum(m), jnp.int32), jnp.zeros((16,), jnp.int32))
        return counts
    totals = lax.fori_loop(0, NCHUNK, hist_step, jnp.zeros((16,), jnp.int32))

    padded = jnp.bitwise_and(totals + (TM - 1), -TM)
    incl = plsc.cumsum(padded)
    seg = incl - padded
    total_padded = jnp.max(incl)

    for i in range(PPW // 16):
        chunk_v[pl.ds(i * 16, 16)] = jnp.full((16,), TRASH, jnp.int32)
    lo = jnp.full((16,), base, jnp.int32)
    hi = jnp.full((16,), base + PPW, jnp.int32)

    def slot_step(i, cur):
        vec = idx_v[pl.ds(i * 16, 16)]
        svec = jnp.zeros((16,), jnp.int32)
        ncur = []
        for e in range(E):
            m = vec == e
            pos = plsc.cumsum(m.astype(jnp.int32))
            cvec = jnp.full((16,), cur[e], jnp.int32)
            svec = jnp.where(m, cvec + pos - 1, svec)
            ncur.append(cur[e] + pos[15])
        ids = lanes + jnp.full((16,), i * 16, jnp.int32)
        keep = jnp.logical_and(svec >= lo, svec < hi)
        plsc.store_scatter(chunk_v, [svec - lo], ids, mask=keep)
        return tuple(ncur)
    lax.fori_loop(0, NCHUNK, slot_step,
                  tuple(seg[e] for e in range(E)), unroll=2)

    pltpu.sync_copy(chunk_v, perm_hbm.at[pl.ds(base, PPW)])

    @pl.when(w == 0)
    def _meta():
        tp = jnp.full((16,), total_padded, jnp.int32)
        tile_starts = lax.iota(jnp.int32, 16) * TM
        valid = (tile_starts < tp).astype(jnp.int32)
        eff = jnp.minimum(tile_starts, tp - TM)
        acc = jnp.full((16,), -1, jnp.int32)
        for e in range(E):
            acc = acc + (eff >= jnp.full((16,), seg[e], jnp.int32)
                         ).astype(jnp.int32)
        meta_v[...] = acc
        pltpu.sync_copy(meta_v, texp_hbm)
        meta_v[...] = valid
        pltpu.sync_copy(meta_v, tvalid_hbm)
        meta_v[...] = tp
        pltpu.sync_copy(meta_v, stats_hbm)


def _sort(idx):
    mesh = plsc.VectorSubcoreMesh(core_axis_name="c", subcore_axis_name="s",
                                  num_cores=NC, num_subcores=NS)
    return pl.kernel(
        _sort_body,
        out_type=(
            jax.ShapeDtypeStruct((P,), jnp.int32),
            jax.ShapeDtypeStruct((16,), jnp.int32),
            jax.ShapeDtypeStruct((16,), jnp.int32),
            jax.ShapeDtypeStruct((16,), jnp.int32),
        ),
        mesh=mesh,
        compiler_params=pltpu.CompilerParams(needs_layout_passes=False),
        scratch_types=[
            pltpu.VMEM((TOKENS,), jnp.int32),
            pltpu.VMEM((PPW,), jnp.int32),
            pltpu.VMEM((16,), jnp.int32),
            pltpu.SemaphoreType.DMA,
        ],
    )(idx)


CH = 32
NCH = SPW // CH
NBUF = 3


def _gather_body(perm_hbm, x_hbm, stats_hbm, xs_hbm,
                 pidx_v, stat_v, ra_v, rb_v, rc_v, gsem, wsem):
    cid = lax.axis_index("c")
    sid = lax.axis_index("s")
    w = sid * NC + cid
    base = w * SPW
    pltpu.sync_copy(stats_hbm, stat_v)
    tp = stat_v[...][0]

    @pl.when(base < tp)
    def _():
        pltpu.sync_copy(perm_hbm.at[pl.ds(base, SPW)], pidx_v)
        for i in range(SPW // 16):
            pidx_v[pl.ds(i * 16, 16)] = jnp.minimum(
                pidx_v[pl.ds(i * 16, 16)], TOKENS - 1)
        bufs = (ra_v, rb_v, rc_v)
        gd = [None] * NCH
        wd = [None] * NCH
        for c in range(min(NBUF, NCH)):
            gd[c] = pltpu.async_copy(
                x_hbm.at[pidx_v.at[pl.ds(c * CH, CH)]], bufs[c % NBUF], gsem)
        for c in range(NCH):
            gd[c].wait()
            wd[c] = pltpu.async_copy(
                bufs[c % NBUF], xs_hbm.at[pl.ds(base + c * CH, CH)], wsem)
            if c + NBUF < NCH:
                wd[c].wait()
                gd[c + NBUF] = pltpu.async_copy(
                    x_hbm.at[pidx_v.at[pl.ds((c + NBUF) * CH, CH)]],
                    bufs[c % NBUF], gsem)
        for c in range(max(0, NCH - NBUF), NCH):
            wd[c].wait()


def _gather(perm, x, stats):
    mesh = plsc.VectorSubcoreMesh(core_axis_name="c", subcore_axis_name="s",
                                  num_cores=NC, num_subcores=NS)
    return pl.kernel(
        _gather_body,
        out_type=jax.ShapeDtypeStruct((P, DIM), jnp.float32),
        mesh=mesh,
        compiler_params=pltpu.CompilerParams(needs_layout_passes=False),
        scratch_types=[
            pltpu.VMEM((SPW,), jnp.int32),
            pltpu.VMEM((16,), jnp.int32),
            pltpu.VMEM((CH, DIM), jnp.float32),
            pltpu.VMEM((CH, DIM), jnp.float32),
            pltpu.VMEM((CH, DIM), jnp.float32),
            pltpu.SemaphoreType.DMA,
            pltpu.SemaphoreType.DMA,
        ],
    )(perm, x, stats)


def _gmm_body(texp_ref, tvalid_ref, xs_ref, ew_ref,
              w1_ref, b1_ref, w3_ref, b3_ref, w2_ref, b2_ref, out_ref):
    t2 = pl.program_id(0)
    k = pl.program_id(1)
    jj = pl.program_id(2)

    for j in range(2):
        @pl.when(jnp.logical_and(jj == j, tvalid_ref[2 * t2 + j] == 1))
        def _compute():
            x = xs_ref[j * TM:(j + 1) * TM, :]
            a = lax.dot_general(x, w1_ref[0], (((1,), (0,)), ((), ())),
                                preferred_element_type=jnp.float32
                                ) + b1_ref[0, 0]
            g = lax.dot_general(x, w3_ref[0], (((1,), (0,)), ((), ())),
                                preferred_element_type=jnp.float32
                                ) + b3_ref[0, 0]
            h = _silu(a) * g
            o = lax.dot_general(h, w2_ref[0], (((1,), (0,)), ((), ())),
                                preferred_element_type=jnp.float32)

            @pl.when(k == 0)
            def _wb():
                out_ref[j * TM:(j + 1) * TM, :] = o + b2_ref[0, 0]

            @pl.when(k != 0)
            def _nb():
                out_ref[j * TM:(j + 1) * TM, :] += o

            @pl.when(k == NK - 1)
            def _sc():
                out_ref[j * TM:(j + 1) * TM, :] *= ew_ref[
                    texp_ref[2 * t2 + j]]


def _gmm(texp, tvalid, xs, expert_weights, w1, b1, w3, b3, w2, b2):
    grid_spec = pltpu.PrefetchScalarGridSpec(
        num_scalar_prefetch=2,
        grid=(NTP // 2, NK, 2),
        in_specs=[
            pl.BlockSpec((2 * TM, DIM), lambda t2, k, j, te, tv: (t2, 0)),
            pl.BlockSpec(memory_space=pltpu.SMEM),
            pl.BlockSpec((1, DIM, TN),
                         lambda t2, k, j, te, tv: (te[2 * t2 + j], 0, k)),
            pl.BlockSpec((1, 1, TN),
                         lambda t2, k, j, te, tv: (te[2 * t2 + j], 0, k)),
            pl.BlockSpec((1, DIM, TN),
                         lambda t2, k, j, te, tv: (te[2 * t2 + j], 0, k)),
            pl.BlockSpec((1, 1, TN),
                         lambda t2, k, j, te, tv: (te[2 * t2 + j], 0, k)),
            pl.BlockSpec((1, TN, DIM),
                         lambda t2, k, j, te, tv: (te[2 * t2 + j], k, 0)),
            pl.BlockSpec((1, 1, DIM),
                         lambda t2, k, j, te, tv: (te[2 * t2 + j], 0, 0)),
        ],
        out_specs=pl.BlockSpec((2 * TM, DIM), lambda t2, k, j, te, tv: (t2, 0)),
    )
    return pl.pallas_call(
        _gmm_body,
        grid_spec=grid_spec,
        out_shape=jax.ShapeDtypeStruct((P, DIM), jnp.float32),
    )(texp, tvalid, xs, expert_weights,
      w1, b1.reshape(E, 1, INTER), w3, b3.reshape(E, 1, INTER),
      w2, b2.reshape(E, 1, DIM))


def _scatter_body(ys_hbm, perm_hbm, stats_hbm, yb_hbm,
                  i0_v, i1_v, i2_v, i3_v, stat_v, ra_v, rb_v, rc_v,
                  lsem, ssem):
    cid = lax.axis_index("c")
    sid = lax.axis_index("s")
    w = sid * NC + cid
    base = w * SPW
    pltpu.sync_copy(stats_hbm, stat_v)
    tp = stat_v[...][0]

    @pl.when(base < tp)
    def _():
        idxs = (i0_v, i1_v, i2_v, i3_v)
        for c in range(NCH):
            pltpu.sync_copy(perm_hbm.at[pl.ds(base + c * CH, CH)], idxs[c])
        bufs = (ra_v, rb_v, rc_v)
        ld = [None] * NCH
        sd = [None] * NCH
        for c in range(min(NBUF, NCH)):
            ld[c] = pltpu.async_copy(
                ys_hbm.at[pl.ds(base + c * CH, CH)], bufs[c % NBUF], lsem)
        for c in range(NCH):
            ld[c].wait()
            sd[c] = pltpu.async_copy(bufs[c % NBUF], yb_hbm.at[idxs[c]], ssem)
            if c + NBUF < NCH:
                sd[c].wait()
                ld[c + NBUF] = pltpu.async_copy(
                    ys_hbm.at[pl.ds(base + (c + NBUF) * CH, CH)],
                    bufs[c % NBUF], lsem)
        for c in range(max(0, NCH - NBUF), NCH):
            sd[c].wait()


def _scatter(y_sorted, perm, stats):
    mesh = plsc.VectorSubcoreMesh(core_axis_name="c", subcore_axis_name="s",
                                  num_cores=NC, num_subcores=NS)
    return pl.kernel(
        _scatter_body,
        out_type=jax.ShapeDtypeStruct((TOKENS + 8, DIM), jnp.float32),
        mesh=mesh,
        compiler_params=pltpu.CompilerParams(needs_layout_passes=False),
        scratch_types=[
            pltpu.VMEM((CH,), jnp.int32),
            pltpu.VMEM((CH,), jnp.int32),
            pltpu.VMEM((CH,), jnp.int32),
            pltpu.VMEM((CH,), jnp.int32),
            pltpu.VMEM((16,), jnp.int32),
            pltpu.VMEM((CH, DIM), jnp.float32),
            pltpu.VMEM((CH, DIM), jnp.float32),
            pltpu.VMEM((CH, DIM), jnp.float32),
            pltpu.SemaphoreType.DMA,
            pltpu.SemaphoreType.DMA,
        ],
    )(y_sorted, perm, stats)


def _shared_body(x_ref, y_ref, ws1_ref, bs1_ref, ws2_ref, bs2_ref, out_ref):
    k = pl.program_id(1)

    @pl.when(k == 0)
    def _init():
        out_ref[...] = y_ref[...] + bs2_ref[...]

    a = lax.dot_general(x_ref[...], ws1_ref[...], (((1,), (0,)), ((), ())),
                        preferred_element_type=jnp.float32) + bs1_ref[...]
    h = _silu(a).astype(jnp.bfloat16)
    out_ref[...] += lax.dot_general(h, ws2_ref[...],
                                    (((1,), (0,)), ((), ())),
                                    preferred_element_type=jnp.float32)


def _shared(x, ybuf, ws1, bs1, ws2, bs2):
    return pl.pallas_call(
        _shared_body,
        grid=(NTS, NKS),
        in_specs=[
            pl.BlockSpec((TM, DIM), lambda t, k: (t, 0)),
            pl.BlockSpec((TM, DIM), lambda t, k: (t, 0)),
            pl.BlockSpec((DIM, TN), lambda t, k: (0, k)),
            pl.BlockSpec((1, TN), lambda t, k: (0, k)),
            pl.BlockSpec((TN, DIM), lambda t, k: (k, 0)),
            pl.BlockSpec((1, DIM), lambda t, k: (0, 0)),
        ],
        out_specs=pl.BlockSpec((TM, DIM), lambda t, k: (t, 0)),
        out_shape=jax.ShapeDtypeStruct((TOKENS, DIM), jnp.float32),
    )(x.astype(jnp.bfloat16), ybuf, ws1.astype(jnp.bfloat16),
      bs1.reshape(1, SH), ws2.astype(jnp.bfloat16), bs2.reshape(1, DIM))


def _add_body(z_ref, y_ref, out_ref):
    out_ref[...] = z_ref[...] + y_ref[...]


def _final_add(z, ybuf):
    return pl.pallas_call(
        _add_body,
        grid=(NTS,),
        in_specs=[
            pl.BlockSpec((TM, DIM), lambda t: (t, 0)),
            pl.BlockSpec((TM, DIM), lambda t: (t, 0)),
        ],
        out_specs=pl.BlockSpec((TM, DIM), lambda t: (t, 0)),
        out_shape=jax.ShapeDtypeStruct((TOKENS, DIM), jnp.float32),
    )(z, ybuf)


def kernel(x, language_token, routing_embeddings, expert_weights,
           w1, b1, w2, b2, w3, b3, ws1, bs1, ws2, bs2):
    idx = _route(language_token, routing_embeddings)
    perm, texp, tvalid, stats = _sort(idx)
    xs = _gather(perm, x, stats)
    ys = _gmm(texp, tvalid, xs, expert_weights, w1, b1, w3, b3, w2, b2)
    ybuf = _scatter(ys, perm, stats)
    return _shared(x, ybuf, ws1, bs1, ws2, bs2)
